# baseline (speedup 1.0000x reference)
import functools

import jax
import jax.numpy as jnp
from jax import lax
from jax.experimental import pallas as pl
from jax.experimental.pallas import tpu as pltpu
from jax.experimental.pallas import tpu_sc as plsc

E = 8
TOPK = 2
D = 768
I = 3072
IS = 1536
T = 2048
N = T * TOPK
BT = 576
NB = -(-N // BT) + E - 1
NPAD = NB * BT
BI = 1536
KI = I // BI
BIS = 768
KIS = IS // BIS
BTS = 2048
HIGH = jax.lax.Precision.HIGHEST


def _cumsum_rows(a, n):
    sh = 1
    while sh < n:
        z = jnp.zeros((sh,) + a.shape[1:], a.dtype)
        a = a + jnp.concatenate([z, a[:n - sh]], axis=0)
        sh *= 2
    return a


def _cumsum_lanes(a, n):
    sh = 1
    while sh < n:
        z = jnp.zeros(a.shape[:-1] + (sh,), a.dtype)
        a = a + jnp.concatenate([z, a[..., :n - sh]], axis=-1)
        sh *= 2
    return a


def _router_body(logits_ref, w_ref, pos0_ref, pos1_ref, be_ref, br_ref,
                 nbr_ref):
    logits = logits_ref[...]
    m = jnp.max(logits, axis=1, keepdims=True)
    p = jnp.exp(logits - m)
    s = p / jnp.sum(p, axis=1, keepdims=True)
    iota = lax.broadcasted_iota(jnp.int32, s.shape, 1)
    m1 = jnp.max(s, axis=1, keepdims=True)
    i1 = jnp.min(jnp.where(s >= m1, iota, E), axis=1, keepdims=True)
    s2 = jnp.where(iota == i1, -jnp.inf, s)
    m2 = jnp.max(s2, axis=1, keepdims=True)
    i2 = jnp.min(jnp.where(s2 >= m2, iota, E), axis=1, keepdims=True)
    w_ref[...] = jnp.concatenate([m1, m2], axis=1)

    oh0 = (iota == i1).astype(jnp.int32)
    oh1 = (iota == i2).astype(jnp.int32)
    csum = _cumsum_rows(oh0 + oh1, T)
    counts = csum[T - 1:T, :]
    excl = csum - (oh0 + oh1)
    rank0 = jnp.sum(jnp.where(oh0 == 1, excl, 0), axis=1, keepdims=True)
    rank1 = jnp.sum(jnp.where(oh1 == 1, excl + oh0, 0), axis=1, keepdims=True)
    bpe = (counts + BT - 1) // BT
    bstart = _cumsum_lanes(bpe, E) - bpe
    nb = jnp.sum(bpe)
    off0 = jnp.sum(jnp.where(oh0 == 1, bstart, 0), axis=1, keepdims=True) * BT
    off1 = jnp.sum(jnp.where(oh1 == 1, bstart, 0), axis=1, keepdims=True) * BT
    pos0_ref[...] = off0 + rank0
    pos1_ref[...] = off1 + rank1

    bidx = lax.broadcasted_iota(jnp.int32, (NB, E), 0)
    bstart_b = jnp.broadcast_to(bstart, (NB, E))
    be_raw = jnp.sum((bidx >= bstart_b).astype(jnp.int32), axis=1,
                     keepdims=True) - 1
    brow = lax.broadcasted_iota(jnp.int32, (NB, 1), 0)
    be_last = jnp.sum(jnp.where(brow == nb - 1, be_raw, 0), axis=0,
                      keepdims=True)
    real = brow < nb
    be_ref[...] = jnp.where(real, be_raw, be_last)
    br_ref[...] = jnp.where(real, brow, nb - 1)
    nbr_ref[...] = jnp.full((1, 1), nb, jnp.int32)


def _router(logits):
    return pl.pallas_call(
        _router_body,
        out_shape=(jax.ShapeDtypeStruct((T, TOPK), jnp.float32),
                   jax.ShapeDtypeStruct((T, 1), jnp.int32),
                   jax.ShapeDtypeStruct((T, 1), jnp.int32),
                   jax.ShapeDtypeStruct((NB, 1), jnp.int32),
                   jax.ShapeDtypeStruct((NB, 1), jnp.int32),
                   jax.ShapeDtypeStruct((1, 1), jnp.int32)),
    )(logits)


_NW = 32
_TPW = T // _NW
_SC_MESH = plsc.VectorSubcoreMesh(core_axis_name="c", subcore_axis_name="s")


@functools.partial(
    pl.kernel, mesh=_SC_MESH,
    out_type=jax.ShapeDtypeStruct((NPAD, D), jnp.float32),
    scratch_types=[
        pltpu.VMEM((_TPW,), jnp.int32),
        pltpu.VMEM((_TPW,), jnp.int32),
        pltpu.VMEM((_TPW, D), jnp.float32),
        pltpu.SemaphoreType.DMA,
        pltpu.SemaphoreType.DMA,
        pltpu.SemaphoreType.DMA,
    ],
)
def _sc_dispatch(x_hbm, pos0_hbm, pos1_hbm, xs_hbm, p0_v, p1_v, rows_v,
                 sem0, sem1, sem2):
    wid = lax.axis_index("s") * 2 + lax.axis_index("c")
    base = wid * _TPW
    l0 = pltpu.async_copy(pos0_hbm.at[pl.ds(base, _TPW)], p0_v, sem0)
    l1 = pltpu.async_copy(pos1_hbm.at[pl.ds(base, _TPW)], p1_v, sem1)
    l2 = pltpu.async_copy(x_hbm.at[pl.ds(base, _TPW)], rows_v, sem2)
    l0.wait()
    l1.wait()
    l2.wait()
    c0 = pltpu.async_copy(rows_v, xs_hbm.at[p0_v], sem0)
    c1 = pltpu.async_copy(rows_v, xs_hbm.at[p1_v], sem1)
    c0.wait()
    c1.wait()


@functools.partial(
    pl.kernel, mesh=_SC_MESH,
    out_type=(jax.ShapeDtypeStruct((T, D), jnp.float32),
              jax.ShapeDtypeStruct((T, D), jnp.float32)),
    scratch_types=[
        pltpu.VMEM((_TPW,), jnp.int32),
        pltpu.VMEM((_TPW,), jnp.int32),
        pltpu.VMEM((_TPW, D), jnp.float32),
        pltpu.VMEM((_TPW, D), jnp.float32),
        pltpu.SemaphoreType.DMA,
        pltpu.SemaphoreType.DMA,
    ],
)
def _sc_unsort(ys_hbm, pos0_hbm, pos1_hbm, yp0_hbm, yp1_hbm, p0_v, p1_v,
               r0_v, r1_v, sem0, sem1):
    wid = lax.axis_index("s") * 2 + lax.axis_index("c")
    base = wid * _TPW
    l0 = pltpu.async_copy(pos0_hbm.at[pl.ds(base, _TPW)], p0_v, sem0)
    l1 = pltpu.async_copy(pos1_hbm.at[pl.ds(base, _TPW)], p1_v, sem1)
    l0.wait()
    l1.wait()
    c0 = pltpu.async_copy(ys_hbm.at[p0_v], r0_v, sem0)
    c1 = pltpu.async_copy(ys_hbm.at[p1_v], r1_v, sem1)
    c0.wait()
    c1.wait()
    s0 = pltpu.async_copy(r0_v, yp0_hbm.at[pl.ds(base, _TPW)], sem0)
    s1 = pltpu.async_copy(r1_v, yp1_hbm.at[pl.ds(base, _TPW)], sem1)
    s0.wait()
    s1.wait()


def _moe_body(be_ref, br_ref, nbr_ref, xs_ref, gw_ref, uw_ref, dw_ref, out_ref):
    b = pl.program_id(0)
    ki = pl.program_id(1)

    @pl.when(b < nbr_ref[0, 0])
    def _():
        xb = xs_ref[...]
        g = lax.dot_general(xb, gw_ref[0], (((1,), (1,)), ((), ())),
                            preferred_element_type=jnp.float32)
        u = lax.dot_general(xb, uw_ref[0], (((1,), (1,)), ((), ())),
                            preferred_element_type=jnp.float32)
        h = g * jax.nn.sigmoid(g) * u
        contrib = lax.dot_general(h, dw_ref[0], (((1,), (1,)), ((), ())),
                                  preferred_element_type=jnp.float32)

        @pl.when(ki == 0)
        def _():
            out_ref[...] = jnp.zeros_like(out_ref)

        out_ref[...] += contrib


def _grouped_mlp(xs, gwb, uwb, dwb, block_expert, block_row, nb_real):
    def ki_sel(b, ki, nbr):
        return jnp.where(b < nbr[0, 0], ki, KI - 1)

    grid_spec = pltpu.PrefetchScalarGridSpec(
        num_scalar_prefetch=3,
        grid=(NB, KI),
        in_specs=[
            pl.BlockSpec((BT, D), lambda b, ki, be, br, nbr: (br[b, 0], 0)),
            pl.BlockSpec((1, BI, D),
                         lambda b, ki, be, br, nbr: (be[b, 0], ki_sel(b, ki, nbr), 0)),
            pl.BlockSpec((1, BI, D),
                         lambda b, ki, be, br, nbr: (be[b, 0], ki_sel(b, ki, nbr), 0)),
            pl.BlockSpec((1, D, BI),
                         lambda b, ki, be, br, nbr: (be[b, 0], 0, ki_sel(b, ki, nbr))),
        ],
        out_specs=pl.BlockSpec((BT, D), lambda b, ki, be, br, nbr: (b, 0)),
    )
    return pl.pallas_call(
        _moe_body,
        grid_spec=grid_spec,
        out_shape=jax.ShapeDtypeStruct((NPAD, D), jnp.float32),
    )(block_expert, block_row, nb_real, xs, gwb, uwb, dwb)


def _shared_body(x_ref, sg_ref, su_ref, sd_ref, out_ref):
    ki = pl.program_id(1)
    xb = x_ref[...]
    g = lax.dot_general(xb, sg_ref[...], (((1,), (1,)), ((), ())),
                        preferred_element_type=jnp.float32)
    u = lax.dot_general(xb, su_ref[...], (((1,), (1,)), ((), ())),
                        preferred_element_type=jnp.float32)
    h = g * jax.nn.sigmoid(g) * u
    contrib = lax.dot_general(h, sd_ref[...], (((1,), (1,)), ((), ())),
                              preferred_element_type=jnp.float32)

    @pl.when(ki == 0)
    def _():
        out_ref[...] = jnp.zeros_like(out_ref)

    out_ref[...] += contrib


def _shared_mlp(x, sg, su, sd):
    return pl.pallas_call(
        _shared_body,
        grid=(T // BTS, KIS),
        in_specs=[
            pl.BlockSpec((BTS, D), lambda tb, ki: (tb, 0)),
            pl.BlockSpec((BIS, D), lambda tb, ki: (ki, 0)),
            pl.BlockSpec((BIS, D), lambda tb, ki: (ki, 0)),
            pl.BlockSpec((D, BIS), lambda tb, ki: (0, ki)),
        ],
        out_specs=pl.BlockSpec((BTS, D), lambda tb, ki: (tb, 0)),
        out_shape=jax.ShapeDtypeStruct((T, D), jnp.float32),
    )(x, sg, su, sd)


def _combine_body(sh_ref, yp0_ref, yp1_ref, tw_ref, out_ref):
    tw = tw_ref[...]
    out_ref[...] = (sh_ref[...] + yp0_ref[...] * tw[:, 0:1]
                    + yp1_ref[...] * tw[:, 1:2])


def _combine(sh, yp0, yp1, topk_w):
    return pl.pallas_call(
        _combine_body,
        grid=(T // BTS,),
        in_specs=[
            pl.BlockSpec((BTS, D), lambda tb: (tb, 0)),
            pl.BlockSpec((BTS, D), lambda tb: (tb, 0)),
            pl.BlockSpec((BTS, D), lambda tb: (tb, 0)),
            pl.BlockSpec((BTS, TOPK), lambda tb: (tb, 0)),
        ],
        out_specs=pl.BlockSpec((BTS, D), lambda tb: (tb, 0)),
        out_shape=jax.ShapeDtypeStruct((T, D), jnp.float32),
    )(sh, yp0, yp1, topk_w)


def kernel(hidden_states, gate_w, gate_proj, up_proj, down_proj,
           sh_gate, sh_up, sh_down):
    b, s, h = hidden_states.shape
    x = hidden_states.reshape(T, D)

    logits = x @ gate_w.T
    topk_w, pos0, pos1, block_expert, block_row, nb_real = _router(logits)
    pos0 = pos0.reshape(T)
    pos1 = pos1.reshape(T)

    xs = _sc_dispatch(x, pos0, pos1)

    ys = _grouped_mlp(xs, gate_proj, up_proj, down_proj,
                      block_expert, block_row, nb_real)

    yp0, yp1 = _sc_unsort(ys, pos0, pos1)

    sh = _shared_mlp(x, sh_gate, sh_up, sh_down)

    out = _combine(sh, yp0, yp1, topk_w)
    return out.reshape(b, s, h)


# speedup vs baseline: 1.0032x; 1.0032x over previous
import functools

import jax
import jax.numpy as jnp
from jax import lax
from jax.experimental import pallas as pl
from jax.experimental.pallas import tpu as pltpu
from jax.experimental.pallas import tpu_sc as plsc

E = 8
TOPK = 2
D = 768
I = 3072
IS = 1536
T = 2048
N = T * TOPK
BT = 576
NB = -(-N // BT) + E - 1
NPAD = NB * BT
BI = 1536
KI = I // BI
BIS = 768
KIS = IS // BIS
BTS = 2048
HIGH = jax.lax.Precision.HIGHEST


def _cumsum_rows(a, n):
    sh = 1
    while sh < n:
        z = jnp.zeros((sh,) + a.shape[1:], a.dtype)
        a = a + jnp.concatenate([z, a[:n - sh]], axis=0)
        sh *= 2
    return a


def _cumsum_lanes(a, n):
    sh = 1
    while sh < n:
        z = jnp.zeros(a.shape[:-1] + (sh,), a.dtype)
        a = a + jnp.concatenate([z, a[..., :n - sh]], axis=-1)
        sh *= 2
    return a


def _router_body(logits_ref, w_ref, pos0_ref, pos1_ref, be_ref, br_ref,
                 nbr_ref):
    logits = logits_ref[...]
    m = jnp.max(logits, axis=1, keepdims=True)
    p = jnp.exp(logits - m)
    s = p / jnp.sum(p, axis=1, keepdims=True)
    iota = lax.broadcasted_iota(jnp.int32, s.shape, 1)
    m1 = jnp.max(s, axis=1, keepdims=True)
    i1 = jnp.min(jnp.where(s >= m1, iota, E), axis=1, keepdims=True)
    s2 = jnp.where(iota == i1, -jnp.inf, s)
    m2 = jnp.max(s2, axis=1, keepdims=True)
    i2 = jnp.min(jnp.where(s2 >= m2, iota, E), axis=1, keepdims=True)
    w_ref[...] = jnp.concatenate([m1, m2], axis=1)

    oh0 = (iota == i1).astype(jnp.int32)
    oh1 = (iota == i2).astype(jnp.int32)
    csum = _cumsum_rows(oh0 + oh1, T)
    counts = csum[T - 1:T, :]
    excl = csum - (oh0 + oh1)
    rank0 = jnp.sum(jnp.where(oh0 == 1, excl, 0), axis=1, keepdims=True)
    rank1 = jnp.sum(jnp.where(oh1 == 1, excl + oh0, 0), axis=1, keepdims=True)
    bpe = (counts + BT - 1) // BT
    bstart = _cumsum_lanes(bpe, E) - bpe
    nb = jnp.sum(bpe)
    off0 = jnp.sum(jnp.where(oh0 == 1, bstart, 0), axis=1, keepdims=True) * BT
    off1 = jnp.sum(jnp.where(oh1 == 1, bstart, 0), axis=1, keepdims=True) * BT
    pos0_ref[...] = off0 + rank0
    pos1_ref[...] = off1 + rank1

    bidx = lax.broadcasted_iota(jnp.int32, (NB, E), 0)
    bstart_b = jnp.broadcast_to(bstart, (NB, E))
    be_raw = jnp.sum((bidx >= bstart_b).astype(jnp.int32), axis=1,
                     keepdims=True) - 1
    brow = lax.broadcasted_iota(jnp.int32, (NB, 1), 0)
    be_last = jnp.sum(jnp.where(brow == nb - 1, be_raw, 0), axis=0,
                      keepdims=True)
    real = brow < nb
    be_ref[...] = jnp.where(real, be_raw, be_last)
    br_ref[...] = jnp.where(real, brow, nb - 1)
    nbr_ref[...] = jnp.full((1, 1), nb, jnp.int32)


def _router(logits):
    return pl.pallas_call(
        _router_body,
        out_shape=(jax.ShapeDtypeStruct((T, TOPK), jnp.float32),
                   jax.ShapeDtypeStruct((T, 1), jnp.int32),
                   jax.ShapeDtypeStruct((T, 1), jnp.int32),
                   jax.ShapeDtypeStruct((NB, 1), jnp.int32),
                   jax.ShapeDtypeStruct((NB, 1), jnp.int32),
                   jax.ShapeDtypeStruct((1, 1), jnp.int32)),
    )(logits)


_NW = 32
_TPW = T // _NW
_SC_MESH = plsc.VectorSubcoreMesh(core_axis_name="c", subcore_axis_name="s")


@functools.partial(
    pl.kernel, mesh=_SC_MESH,
    out_type=jax.ShapeDtypeStruct((NPAD, D), jnp.float32),
    scratch_types=[
        pltpu.VMEM((_TPW,), jnp.int32),
        pltpu.VMEM((_TPW,), jnp.int32),
        pltpu.VMEM((_TPW, D), jnp.float32),
        pltpu.SemaphoreType.DMA,
        pltpu.SemaphoreType.DMA,
    ],
)
def _sc_dispatch(x_hbm, pos0_hbm, pos1_hbm, xs_hbm, p0_v, p1_v, rows_v,
                 sem0, sem1):
    wid = lax.axis_index("s") * 2 + lax.axis_index("c")
    base = wid * _TPW
    pltpu.sync_copy(pos0_hbm.at[pl.ds(base, _TPW)], p0_v)
    pltpu.sync_copy(pos1_hbm.at[pl.ds(base, _TPW)], p1_v)
    pltpu.sync_copy(x_hbm.at[pl.ds(base, _TPW)], rows_v)
    c0 = pltpu.async_copy(rows_v, xs_hbm.at[p0_v], sem0)
    c1 = pltpu.async_copy(rows_v, xs_hbm.at[p1_v], sem1)
    c0.wait()
    c1.wait()


@functools.partial(
    pl.kernel, mesh=_SC_MESH,
    out_type=(jax.ShapeDtypeStruct((T, D), jnp.float32),
              jax.ShapeDtypeStruct((T, D), jnp.float32)),
    scratch_types=[
        pltpu.VMEM((_TPW,), jnp.int32),
        pltpu.VMEM((_TPW,), jnp.int32),
        pltpu.VMEM((_TPW, D), jnp.float32),
        pltpu.VMEM((_TPW, D), jnp.float32),
        pltpu.SemaphoreType.DMA,
        pltpu.SemaphoreType.DMA,
    ],
)
def _sc_unsort(ys_hbm, pos0_hbm, pos1_hbm, yp0_hbm, yp1_hbm, p0_v, p1_v,
               r0_v, r1_v, sem0, sem1):
    wid = lax.axis_index("s") * 2 + lax.axis_index("c")
    base = wid * _TPW
    pltpu.sync_copy(pos0_hbm.at[pl.ds(base, _TPW)], p0_v)
    pltpu.sync_copy(pos1_hbm.at[pl.ds(base, _TPW)], p1_v)
    c0 = pltpu.async_copy(ys_hbm.at[p0_v], r0_v, sem0)
    c1 = pltpu.async_copy(ys_hbm.at[p1_v], r1_v, sem1)
    c0.wait()
    c1.wait()
    pltpu.sync_copy(r0_v, yp0_hbm.at[pl.ds(base, _TPW)])
    pltpu.sync_copy(r1_v, yp1_hbm.at[pl.ds(base, _TPW)])


def _moe_body(be_ref, br_ref, nbr_ref, xs_ref, gw_ref, uw_ref, dw_ref, out_ref):
    b = pl.program_id(0)
    ki = pl.program_id(1)

    @pl.when(b < nbr_ref[0, 0])
    def _():
        xb = xs_ref[...]
        g = lax.dot_general(xb, gw_ref[0], (((1,), (1,)), ((), ())),
                            preferred_element_type=jnp.float32)
        u = lax.dot_general(xb, uw_ref[0], (((1,), (1,)), ((), ())),
                            preferred_element_type=jnp.float32)
        h = g * jax.nn.sigmoid(g) * u
        contrib = lax.dot_general(h, dw_ref[0], (((1,), (1,)), ((), ())),
                                  preferred_element_type=jnp.float32)

        @pl.when(ki == 0)
        def _():
            out_ref[...] = jnp.zeros_like(out_ref)

        out_ref[...] += contrib


def _grouped_mlp(xs, gwb, uwb, dwb, block_expert, block_row, nb_real):
    def ki_sel(b, ki, nbr):
        return jnp.where(b < nbr[0, 0], ki, KI - 1)

    grid_spec = pltpu.PrefetchScalarGridSpec(
        num_scalar_prefetch=3,
        grid=(NB, KI),
        in_specs=[
            pl.BlockSpec((BT, D), lambda b, ki, be, br, nbr: (br[b, 0], 0)),
            pl.BlockSpec((1, BI, D),
                         lambda b, ki, be, br, nbr: (be[b, 0], ki_sel(b, ki, nbr), 0)),
            pl.BlockSpec((1, BI, D),
                         lambda b, ki, be, br, nbr: (be[b, 0], ki_sel(b, ki, nbr), 0)),
            pl.BlockSpec((1, D, BI),
                         lambda b, ki, be, br, nbr: (be[b, 0], 0, ki_sel(b, ki, nbr))),
        ],
        out_specs=pl.BlockSpec((BT, D), lambda b, ki, be, br, nbr: (b, 0)),
    )
    return pl.pallas_call(
        _moe_body,
        grid_spec=grid_spec,
        out_shape=jax.ShapeDtypeStruct((NPAD, D), jnp.float32),
    )(block_expert, block_row, nb_real, xs, gwb, uwb, dwb)


def _shared_body(x_ref, sg_ref, su_ref, sd_ref, out_ref):
    ki = pl.program_id(1)
    xb = x_ref[...]
    g = lax.dot_general(xb, sg_ref[...], (((1,), (1,)), ((), ())),
                        preferred_element_type=jnp.float32)
    u = lax.dot_general(xb, su_ref[...], (((1,), (1,)), ((), ())),
                        preferred_element_type=jnp.float32)
    h = g * jax.nn.sigmoid(g) * u
    contrib = lax.dot_general(h, sd_ref[...], (((1,), (1,)), ((), ())),
                              preferred_element_type=jnp.float32)

    @pl.when(ki == 0)
    def _():
        out_ref[...] = jnp.zeros_like(out_ref)

    out_ref[...] += contrib


def _shared_mlp(x, sg, su, sd):
    return pl.pallas_call(
        _shared_body,
        grid=(T // BTS, KIS),
        in_specs=[
            pl.BlockSpec((BTS, D), lambda tb, ki: (tb, 0)),
            pl.BlockSpec((BIS, D), lambda tb, ki: (ki, 0)),
            pl.BlockSpec((BIS, D), lambda tb, ki: (ki, 0)),
            pl.BlockSpec((D, BIS), lambda tb, ki: (0, ki)),
        ],
        out_specs=pl.BlockSpec((BTS, D), lambda tb, ki: (tb, 0)),
        out_shape=jax.ShapeDtypeStruct((T, D), jnp.float32),
    )(x, sg, su, sd)


def _combine_body(sh_ref, yp0_ref, yp1_ref, tw_ref, out_ref):
    tw = tw_ref[...]
    out_ref[...] = (sh_ref[...] + yp0_ref[...] * tw[:, 0:1]
                    + yp1_ref[...] * tw[:, 1:2])


def _combine(sh, yp0, yp1, topk_w):
    return pl.pallas_call(
        _combine_body,
        grid=(T // BTS,),
        in_specs=[
            pl.BlockSpec((BTS, D), lambda tb: (tb, 0)),
            pl.BlockSpec((BTS, D), lambda tb: (tb, 0)),
            pl.BlockSpec((BTS, D), lambda tb: (tb, 0)),
            pl.BlockSpec((BTS, TOPK), lambda tb: (tb, 0)),
        ],
        out_specs=pl.BlockSpec((BTS, D), lambda tb: (tb, 0)),
        out_shape=jax.ShapeDtypeStruct((T, D), jnp.float32),
    )(sh, yp0, yp1, topk_w)


def kernel(hidden_states, gate_w, gate_proj, up_proj, down_proj,
           sh_gate, sh_up, sh_down):
    b, s, h = hidden_states.shape
    x = hidden_states.reshape(T, D)

    logits = x @ gate_w.T
    topk_w, pos0, pos1, block_expert, block_row, nb_real = _router(logits)
    pos0 = pos0.reshape(T)
    pos1 = pos1.reshape(T)

    xs = _sc_dispatch(x, pos0, pos1)

    ys = _grouped_mlp(xs, gate_proj, up_proj, down_proj,
                      block_expert, block_row, nb_real)

    yp0, yp1 = _sc_unsort(ys, pos0, pos1)

    sh = _shared_mlp(x, sh_gate, sh_up, sh_down)

    out = _combine(sh, yp0, yp1, topk_w)
    return out.reshape(b, s, h)


# speedup vs baseline: 1.0040x; 1.0008x over previous
import functools

import jax
import jax.numpy as jnp
from jax import lax
from jax.experimental import pallas as pl
from jax.experimental.pallas import tpu as pltpu
from jax.experimental.pallas import tpu_sc as plsc

E = 8
TOPK = 2
D = 768
I = 3072
IS = 1536
T = 2048
N = T * TOPK
BT = 576
NB = -(-N // BT) + E - 1
NPAD = NB * BT
BI = 1536
KI = I // BI
BIS = 768
KIS = IS // BIS
BTS = 2048


def _cumsum_rows(a, n):
    sh = 1
    while sh < n:
        z = jnp.zeros((sh,) + a.shape[1:], a.dtype)
        a = a + jnp.concatenate([z, a[:n - sh]], axis=0)
        sh *= 2
    return a


def _cumsum_lanes(a, n):
    sh = 1
    while sh < n:
        z = jnp.zeros(a.shape[:-1] + (sh,), a.dtype)
        a = a + jnp.concatenate([z, a[..., :n - sh]], axis=-1)
        sh *= 2
    return a


def _router_body(logits_ref, w_ref, pos0_ref, pos1_ref, be_ref, br_ref,
                 nbr_ref):
    logits = logits_ref[...]
    m = jnp.max(logits, axis=1, keepdims=True)
    p = jnp.exp(logits - m)
    s = p / jnp.sum(p, axis=1, keepdims=True)
    iota = lax.broadcasted_iota(jnp.int32, s.shape, 1)
    m1 = jnp.max(s, axis=1, keepdims=True)
    i1 = jnp.min(jnp.where(s >= m1, iota, E), axis=1, keepdims=True)
    s2 = jnp.where(iota == i1, -jnp.inf, s)
    m2 = jnp.max(s2, axis=1, keepdims=True)
    i2 = jnp.min(jnp.where(s2 >= m2, iota, E), axis=1, keepdims=True)
    w_ref[...] = jnp.concatenate([m1, m2], axis=1)

    oh0 = (iota == i1).astype(jnp.int32)
    oh1 = (iota == i2).astype(jnp.int32)
    csum = _cumsum_rows(oh0 + oh1, T)
    counts = csum[T - 1:T, :]
    excl = csum - (oh0 + oh1)
    rank0 = jnp.sum(jnp.where(oh0 == 1, excl, 0), axis=1, keepdims=True)
    rank1 = jnp.sum(jnp.where(oh1 == 1, excl + oh0, 0), axis=1, keepdims=True)
    bpe = (counts + BT - 1) // BT
    bstart = _cumsum_lanes(bpe, E) - bpe
    nb = jnp.sum(bpe)
    off0 = jnp.sum(jnp.where(oh0 == 1, bstart, 0), axis=1, keepdims=True) * BT
    off1 = jnp.sum(jnp.where(oh1 == 1, bstart, 0), axis=1, keepdims=True) * BT
    pos0_ref[...] = off0 + rank0
    pos1_ref[...] = off1 + rank1

    bidx = lax.broadcasted_iota(jnp.int32, (NB, E), 0)
    bstart_b = jnp.broadcast_to(bstart, (NB, E))
    be_raw = jnp.sum((bidx >= bstart_b).astype(jnp.int32), axis=1,
                     keepdims=True) - 1
    brow = lax.broadcasted_iota(jnp.int32, (NB, 1), 0)
    be_last = jnp.sum(jnp.where(brow == nb - 1, be_raw, 0), axis=0,
                      keepdims=True)
    real = brow < nb
    be_ref[...] = jnp.where(real, be_raw, be_last)
    br_ref[...] = jnp.where(real, brow, nb - 1)
    nbr_ref[...] = jnp.full((1, 1), nb, jnp.int32)


def _router(logits):
    return pl.pallas_call(
        _router_body,
        out_shape=(jax.ShapeDtypeStruct((T, TOPK), jnp.float32),
                   jax.ShapeDtypeStruct((T, 1), jnp.int32),
                   jax.ShapeDtypeStruct((T, 1), jnp.int32),
                   jax.ShapeDtypeStruct((NB, 1), jnp.int32),
                   jax.ShapeDtypeStruct((NB, 1), jnp.int32),
                   jax.ShapeDtypeStruct((1, 1), jnp.int32)),
    )(logits)


_NW = 32
_TPW = T // _NW
_SC_MESH = plsc.VectorSubcoreMesh(core_axis_name="c", subcore_axis_name="s")


@functools.partial(
    pl.kernel, mesh=_SC_MESH,
    out_type=jax.ShapeDtypeStruct((NPAD, D), jnp.float32),
    scratch_types=[
        pltpu.VMEM((_TPW,), jnp.int32),
        pltpu.VMEM((_TPW,), jnp.int32),
        pltpu.VMEM((_TPW, D), jnp.float32),
        pltpu.SemaphoreType.DMA,
        pltpu.SemaphoreType.DMA,
    ],
)
def _sc_dispatch(x_hbm, pos0_hbm, pos1_hbm, xs_hbm, p0_v, p1_v, rows_v,
                 sem0, sem1):
    wid = lax.axis_index("s") * 2 + lax.axis_index("c")
    base = wid * _TPW
    pltpu.sync_copy(pos0_hbm.at[pl.ds(base, _TPW)], p0_v)
    pltpu.sync_copy(pos1_hbm.at[pl.ds(base, _TPW)], p1_v)
    pltpu.sync_copy(x_hbm.at[pl.ds(base, _TPW)], rows_v)
    c0 = pltpu.async_copy(rows_v, xs_hbm.at[p0_v], sem0)
    c1 = pltpu.async_copy(rows_v, xs_hbm.at[p1_v], sem1)
    c0.wait()
    c1.wait()


@functools.partial(
    pl.kernel, mesh=_SC_MESH,
    out_type=(jax.ShapeDtypeStruct((T, D), jnp.float32),
              jax.ShapeDtypeStruct((T, D), jnp.float32)),
    scratch_types=[
        pltpu.VMEM((_TPW,), jnp.int32),
        pltpu.VMEM((_TPW,), jnp.int32),
        pltpu.VMEM((_TPW, D), jnp.float32),
        pltpu.VMEM((_TPW, D), jnp.float32),
        pltpu.SemaphoreType.DMA,
        pltpu.SemaphoreType.DMA,
    ],
)
def _sc_unsort(ys_hbm, pos0_hbm, pos1_hbm, yp0_hbm, yp1_hbm, p0_v, p1_v,
               r0_v, r1_v, sem0, sem1):
    wid = lax.axis_index("s") * 2 + lax.axis_index("c")
    base = wid * _TPW
    pltpu.sync_copy(pos0_hbm.at[pl.ds(base, _TPW)], p0_v)
    pltpu.sync_copy(pos1_hbm.at[pl.ds(base, _TPW)], p1_v)
    c0 = pltpu.async_copy(ys_hbm.at[p0_v], r0_v, sem0)
    c1 = pltpu.async_copy(ys_hbm.at[p1_v], r1_v, sem1)
    c0.wait()
    c1.wait()
    pltpu.sync_copy(r0_v, yp0_hbm.at[pl.ds(base, _TPW)])
    pltpu.sync_copy(r1_v, yp1_hbm.at[pl.ds(base, _TPW)])


def _moe_body(be_ref, br_ref, nbr_ref, xs_ref, gw_ref, uw_ref, dw_ref, out_ref):
    b = pl.program_id(0)
    ki = pl.program_id(1)

    @pl.when(b < nbr_ref[0, 0])
    def _():
        xb = xs_ref[...]
        g = lax.dot_general(xb, gw_ref[0], (((1,), (1,)), ((), ())),
                            preferred_element_type=jnp.float32)
        u = lax.dot_general(xb, uw_ref[0], (((1,), (1,)), ((), ())),
                            preferred_element_type=jnp.float32)
        h = g * jax.nn.sigmoid(g) * u
        contrib = lax.dot_general(h, dw_ref[0], (((1,), (1,)), ((), ())),
                                  preferred_element_type=jnp.float32)

        @pl.when(ki == 0)
        def _():
            out_ref[...] = jnp.zeros_like(out_ref)

        out_ref[...] += contrib


def _grouped_mlp(xs, gwb, uwb, dwb, block_expert, block_row, nb_real):
    def ki_sel(b, ki, nbr):
        return jnp.where(b < nbr[0, 0], ki, KI - 1)

    grid_spec = pltpu.PrefetchScalarGridSpec(
        num_scalar_prefetch=3,
        grid=(NB, KI),
        in_specs=[
            pl.BlockSpec((BT, D), lambda b, ki, be, br, nbr: (br[b, 0], 0)),
            pl.BlockSpec((1, BI, D),
                         lambda b, ki, be, br, nbr: (be[b, 0], ki_sel(b, ki, nbr), 0)),
            pl.BlockSpec((1, BI, D),
                         lambda b, ki, be, br, nbr: (be[b, 0], ki_sel(b, ki, nbr), 0)),
            pl.BlockSpec((1, D, BI),
                         lambda b, ki, be, br, nbr: (be[b, 0], 0, ki_sel(b, ki, nbr))),
        ],
        out_specs=pl.BlockSpec((BT, D), lambda b, ki, be, br, nbr: (b, 0)),
    )
    return pl.pallas_call(
        _moe_body,
        grid_spec=grid_spec,
        out_shape=jax.ShapeDtypeStruct((NPAD, D), jnp.float32),
    )(block_expert, block_row, nb_real, xs, gwb, uwb, dwb)


def _shared_body(x_ref, sg_ref, su_ref, sd_ref, out_ref):
    ki = pl.program_id(1)
    xb = x_ref[...]
    g = lax.dot_general(xb, sg_ref[...], (((1,), (1,)), ((), ())),
                        preferred_element_type=jnp.float32)
    u = lax.dot_general(xb, su_ref[...], (((1,), (1,)), ((), ())),
                        preferred_element_type=jnp.float32)
    h = g * jax.nn.sigmoid(g) * u
    contrib = lax.dot_general(h, sd_ref[...], (((1,), (1,)), ((), ())),
                              preferred_element_type=jnp.float32)

    @pl.when(ki == 0)
    def _():
        out_ref[...] = jnp.zeros_like(out_ref)

    out_ref[...] += contrib


def _shared_mlp(x, sg, su, sd):
    return pl.pallas_call(
        _shared_body,
        grid=(T // BTS, KIS),
        in_specs=[
            pl.BlockSpec((BTS, D), lambda tb, ki: (tb, 0)),
            pl.BlockSpec((BIS, D), lambda tb, ki: (ki, 0)),
            pl.BlockSpec((BIS, D), lambda tb, ki: (ki, 0)),
            pl.BlockSpec((D, BIS), lambda tb, ki: (0, ki)),
        ],
        out_specs=pl.BlockSpec((BTS, D), lambda tb, ki: (tb, 0)),
        out_shape=jax.ShapeDtypeStruct((T, D), jnp.float32),
    )(x, sg, su, sd)


def _combine_body(sh_ref, yp0_ref, yp1_ref, tw_ref, out_ref):
    tw = tw_ref[...]
    out_ref[...] = (sh_ref[...] + yp0_ref[...] * tw[:, 0:1]
                    + yp1_ref[...] * tw[:, 1:2])


def _combine(sh, yp0, yp1, topk_w):
    return pl.pallas_call(
        _combine_body,
        grid=(T // BTS,),
        in_specs=[
            pl.BlockSpec((BTS, D), lambda tb: (tb, 0)),
            pl.BlockSpec((BTS, D), lambda tb: (tb, 0)),
            pl.BlockSpec((BTS, D), lambda tb: (tb, 0)),
            pl.BlockSpec((BTS, TOPK), lambda tb: (tb, 0)),
        ],
        out_specs=pl.BlockSpec((BTS, D), lambda tb: (tb, 0)),
        out_shape=jax.ShapeDtypeStruct((T, D), jnp.float32),
    )(sh, yp0, yp1, topk_w)


def kernel(hidden_states, gate_w, gate_proj, up_proj, down_proj,
           sh_gate, sh_up, sh_down):
    b, s, h = hidden_states.shape
    x = hidden_states.reshape(T, D)

    logits = x @ gate_w.T
    topk_w, pos0, pos1, block_expert, block_row, nb_real = _router(logits)
    pos0 = pos0.reshape(T)
    pos1 = pos1.reshape(T)

    xs = _sc_dispatch(x, pos0, pos1)

    ys = _grouped_mlp(xs, gate_proj, up_proj, down_proj,
                      block_expert, block_row, nb_real)

    yp0, yp1 = _sc_unsort(ys, pos0, pos1)

    sh = _shared_mlp(x, sh_gate, sh_up, sh_down)

    out = _combine(sh, yp0, yp1, topk_w)
    return out.reshape(b, s, h)
